# baseline (speedup 1.0000x reference)
import functools

import jax
import jax.numpy as jnp
from jax import lax
from jax.experimental import pallas as pl
from jax.experimental.pallas import tpu as pltpu
from jax.experimental.pallas import tpu_sc as plsc

NC = 2
NS = 16
NW = NC * NS
L = 16
G = 128
GR = 16
GS = 4


def _make_sc_partial(n_chunks, B, D):
    nj = D // L
    NB = 3
    mesh = plsc.VectorSubcoreMesh(core_axis_name="c", subcore_axis_name="s")

    @functools.partial(
        pl.kernel,
        out_type=jax.ShapeDtypeStruct((NW, B, D), jnp.float32),
        mesh=mesh,
        scratch_types=[
            pltpu.VMEM((n_chunks, G), jnp.int32),
            pltpu.VMEM((n_chunks + 1, G), jnp.int32),
            pltpu.VMEM((2 * L,), jnp.int32),
            [pltpu.VMEM((G, D), jnp.float32) for _ in range(NB)],
            pltpu.VMEM((B, D), jnp.float32),
            [pltpu.SemaphoreType.DMA for _ in range(NB)],
        ],
    )
    def sc_partial(x_hbm, bidx_hbm, segfl_hbm, table_hbm, out_hbm,
                   idx_v, seg_v, segfl_v, rows_bufs, acc_v, sems):
        c = lax.axis_index("c")
        s = lax.axis_index("s")
        wid = s * NC + c

        pltpu.sync_copy(x_hbm.at[wid], idx_v)
        for g in range(n_chunks):
            pltpu.sync_copy(bidx_hbm.at[wid].at[g], seg_v.at[g])
        pltpu.sync_copy(segfl_hbm.at[wid], segfl_v)
        segf = segfl_v[pl.ds(0, L)]
        segl = segfl_v[pl.ds(L, L)]

        def fire(g):
            return pltpu.async_copy(table_hbm.at[idx_v.at[g]],
                                    rows_bufs[g % NB], sems[g % NB])

        copies = {g: fire(g) for g in range(min(NB, n_chunks))}

        zvec = jnp.zeros((L,), jnp.float32)

        def zero_row(i, carry):
            for j in range(nj):
                acc_v[i, pl.ds(L * j, L)] = zvec
            return carry

        lax.fori_loop(0, B, zero_row, 0)

        for g in range(n_chunks):
            rows_v = rows_bufs[g % NB]
            r0 = segf[g]
            r1 = segl[g]
            copies[g].wait()

            def fast_group(k, carry, rows_v=rows_v, r0=r0):
                i0 = k * GR
                ts = [rows_v[i0, pl.ds(L * j, L)] for j in range(nj)]
                for l in range(1, GR):
                    for j in range(nj):
                        ts[j] = ts[j] + rows_v[i0 + l, pl.ds(L * j, L)]
                for j in range(nj):
                    plsc.addupdate(acc_v.at[r0, pl.ds(L * j, L)], ts[j])
                return carry

            def slow_group(k, carry, rows_v=rows_v, seg_row=g):
                i0 = k * GS
                segs = seg_v[seg_row, pl.ds(i0, L)]
                for l in range(GS):
                    r = segs[l]
                    for j in range(nj):
                        plsc.addupdate(acc_v.at[r, pl.ds(L * j, L)],
                                       rows_v[i0 + l, pl.ds(L * j, L)])
                return carry

            @pl.when(r0 == r1)
            def _():
                lax.fori_loop(0, G // GR, fast_group, 0)

            @pl.when(r0 != r1)
            def _():
                lax.fori_loop(0, G // GS, slow_group, 0)

            if g + NB < n_chunks:
                copies[g + NB] = fire(g + NB)

        pltpu.sync_copy(acc_v, out_hbm.at[wid])

    return sc_partial


def _tc_head(p_ref, w_ref, b_ref, o_ref):
    u = jnp.sum(p_ref[...], axis=0)
    r = jnp.maximum(u, 0.0)
    o_ref[...] = jnp.sum(r * w_ref[...], axis=1, keepdims=True) + b_ref[...]


def kernel(x, item_starts, batch_idx, batch_len, tableA, tableC, head_w, head_b):
    del item_starts, tableA
    N = x.shape[0]
    B = batch_len.shape[0]
    D = tableC.shape[1]
    span = NW * G
    assert N % span == 0
    n_chunks = N // span
    assert n_chunks <= L

    segf = batch_idx[0::G].reshape(NW, n_chunks)
    segl = batch_idx[G - 1::G].reshape(NW, n_chunks)
    pad = jnp.zeros((NW, L - n_chunks), jnp.int32)
    segfl = jnp.concatenate([segf, pad, segl, pad], axis=1)

    partial = _make_sc_partial(n_chunks, B, D)(
        x.reshape(NW, n_chunks, G), batch_idx.reshape(NW, n_chunks, G),
        segfl, tableC)

    out = pl.pallas_call(
        _tc_head,
        out_shape=jax.ShapeDtypeStruct((B, 1), jnp.float32),
    )(partial, head_w, head_b.reshape(1, 1))
    return out


# speedup vs baseline: 1.4046x; 1.4046x over previous
import functools

import jax
import jax.numpy as jnp
from jax import lax
from jax.experimental import pallas as pl
from jax.experimental.pallas import tpu as pltpu
from jax.experimental.pallas import tpu_sc as plsc

NC = 2
NS = 16
NW = NC * NS
L = 16
G = 128
GR = 8
GS = 8


def _make_sc_partial(n_chunks, B, D):
    nj = D // L
    NB = 3
    mesh = plsc.VectorSubcoreMesh(core_axis_name="c", subcore_axis_name="s")

    @functools.partial(
        pl.kernel,
        out_type=jax.ShapeDtypeStruct((NW, B, D), jnp.float32),
        mesh=mesh,
        scratch_types=[
            pltpu.VMEM((n_chunks, G), jnp.int32),
            pltpu.VMEM((n_chunks + 1, G), jnp.int32),
            pltpu.VMEM((2 * L,), jnp.int32),
            [pltpu.VMEM((G, D), jnp.float32) for _ in range(NB)],
            pltpu.VMEM((B, D), jnp.float32),
            [pltpu.SemaphoreType.DMA for _ in range(NB)],
        ],
    )
    def sc_partial(x_hbm, bidx_hbm, segfl_hbm, table_hbm, out_hbm,
                   idx_v, seg_v, segfl_v, rows_bufs, acc_v, sems):
        c = lax.axis_index("c")
        s = lax.axis_index("s")
        wid = s * NC + c

        pltpu.sync_copy(x_hbm.at[wid], idx_v)
        pltpu.sync_copy(bidx_hbm.at[wid], seg_v.at[pl.ds(0, n_chunks)])
        pltpu.sync_copy(segfl_hbm.at[wid], segfl_v)
        segf = segfl_v[pl.ds(0, L)]
        segl = segfl_v[pl.ds(L, L)]

        def fire(g):
            return pltpu.async_copy(table_hbm.at[idx_v.at[g]],
                                    rows_bufs[g % NB], sems[g % NB])

        copies = {g: fire(g) for g in range(min(NB, n_chunks))}

        zvec = jnp.zeros((L,), jnp.float32)

        def zero_row(i, carry):
            for j in range(nj):
                acc_v[i, pl.ds(L * j, L)] = zvec
            return carry

        lax.fori_loop(0, B, zero_row, 0)

        for g in range(n_chunks):
            rows_v = rows_bufs[g % NB]
            r0 = segf[g]
            r1 = segl[g]
            copies[g].wait()

            def fast_group(k, ts, rows_v=rows_v):
                i0 = k * GR
                out = list(ts)
                for l in range(GR):
                    for j in range(nj):
                        out[j] = out[j] + rows_v[i0 + l, pl.ds(L * j, L)]
                return tuple(out)

            def slow_group(k, carry, rows_v=rows_v, seg_row=g):
                i0 = k * GS
                segs = seg_v[seg_row, pl.ds(i0, L)]
                for l in range(GS):
                    r = segs[l]
                    for j in range(nj):
                        plsc.addupdate(acc_v.at[r, pl.ds(L * j, L)],
                                       rows_v[i0 + l, pl.ds(L * j, L)])
                return carry

            @pl.when(r0 == r1)
            def _():
                ts0 = tuple(jnp.zeros((L,), jnp.float32) for _ in range(nj))
                ts = lax.fori_loop(0, G // GR, fast_group, ts0)
                for j in range(nj):
                    plsc.addupdate(acc_v.at[r0, pl.ds(L * j, L)], ts[j])

            @pl.when(r0 != r1)
            def _():
                lax.fori_loop(0, G // GS, slow_group, 0)

            if g + NB < n_chunks:
                copies[g + NB] = fire(g + NB)

        pltpu.sync_copy(acc_v, out_hbm.at[wid])

    return sc_partial


def _tc_head(p_ref, w_ref, b_ref, o_ref):
    u = jnp.sum(p_ref[...], axis=0)
    r = jnp.maximum(u, 0.0)
    o_ref[...] = jnp.sum(r * w_ref[...], axis=1, keepdims=True) + b_ref[...]


def kernel(x, item_starts, batch_idx, batch_len, tableA, tableC, head_w, head_b):
    del item_starts, tableA
    N = x.shape[0]
    B = batch_len.shape[0]
    D = tableC.shape[1]
    span = NW * G
    assert N % span == 0
    n_chunks = N // span
    assert n_chunks <= L

    segf = batch_idx[0::G].reshape(NW, n_chunks)
    segl = batch_idx[G - 1::G].reshape(NW, n_chunks)
    pad = jnp.zeros((NW, L - n_chunks), jnp.int32)
    segfl = jnp.concatenate([segf, pad, segl, pad], axis=1)

    partial = _make_sc_partial(n_chunks, B, D)(
        x.reshape(NW, n_chunks, G), batch_idx.reshape(NW, n_chunks, G),
        segfl, tableC)

    out = pl.pallas_call(
        _tc_head,
        out_shape=jax.ShapeDtypeStruct((B, 1), jnp.float32),
    )(partial, head_w, head_b.reshape(1, 1))
    return out
